# Initial kernel scaffold; baseline (speedup 1.0000x reference)
import jax
import jax.numpy as jnp
from jax.experimental import pallas as pl


def kernel(cls, bbox_2d, gts, anchors, gt_labels):
    raise NotImplementedError("write your pallas kernel here")


# speedup vs baseline: 61.6014x; 61.6014x over previous
import jax
import jax.numpy as jnp
from jax.experimental import pallas as pl
from jax.experimental.pallas import tpu as pltpu

H, W = 64, 220
A = 9
N = H * W * A
ROWS, LANES = 990, 128
G = 32
B = 4
STRIDE = 8.0
FG_T, IGN_T = 0.5, 0.4
STDS = (0.1, 0.1, 0.2, 0.2)


def _body(cls_ref, box_ref, gts_ref, lbl_ref, anc_ref, out_ref):
    i = pl.program_id(0)

    r = jax.lax.broadcasted_iota(jnp.int32, (ROWS, LANES), 0)
    l = jax.lax.broadcasted_iota(jnp.int32, (ROWS, LANES), 1)
    n = r * LANES + l
    a = n % A
    cell = n // A
    wi = cell % W
    hi = cell // W
    gx = wi.astype(jnp.float32) * STRIDE
    gy = hi.astype(jnp.float32) * STRIDE

    ax1 = jnp.zeros((ROWS, LANES), jnp.float32)
    ay1 = jnp.zeros((ROWS, LANES), jnp.float32)
    ax2 = jnp.zeros((ROWS, LANES), jnp.float32)
    ay2 = jnp.zeros((ROWS, LANES), jnp.float32)
    for k in range(A):
        sel = a == k
        ax1 = jnp.where(sel, anc_ref[k, 0], ax1)
        ay1 = jnp.where(sel, anc_ref[k, 1], ay1)
        ax2 = jnp.where(sel, anc_ref[k, 2], ax2)
        ay2 = jnp.where(sel, anc_ref[k, 3], ay2)

    x1 = gx + ax1
    y1 = gy + ay1
    x2 = gx + ax2
    y2 = gy + ay2
    aa = (x2 - x1) * (y2 - y1)
    rw = x2 - x1 + 1.0
    rh = y2 - y1 + 1.0
    rcx = x1 + 0.5 * rw
    rcy = y1 + 0.5 * rh

    gb = []
    for g in range(G):
        gb.append((gts_ref[i, g, 0], gts_ref[i, g, 1],
                   gts_ref[i, g, 2], gts_ref[i, g, 3], lbl_ref[i, g]))

    BIG = jnp.int32(1 << 30)
    best = None
    best_inds = []
    for g in range(G):
        gx1, gy1, gx2, gy2, lblg = gb[g]
        ab = (gx2 - gx1) * (gy2 - gy1)
        iw = jnp.maximum(jnp.minimum(x2, gx2) - jnp.maximum(x1, gx1), 0.0)
        ih = jnp.maximum(jnp.minimum(y2, gy2) - jnp.maximum(y1, gy1), 0.0)
        inter = iw * ih
        iou = inter / jnp.maximum(aa + ab - inter, 1e-8)
        mg = jnp.max(iou)
        best_inds.append(jnp.min(jnp.where(iou >= mg, n, BIG)))
        if g == 0:
            best = iou
            labv = jnp.full((ROWS, LANES), lblg, jnp.int32)
            mx1 = jnp.full((ROWS, LANES), gx1, jnp.float32)
            my1 = jnp.full((ROWS, LANES), gy1, jnp.float32)
            mx2 = jnp.full((ROWS, LANES), gx2, jnp.float32)
            my2 = jnp.full((ROWS, LANES), gy2, jnp.float32)
        else:
            upd = iou > best
            best = jnp.where(upd, iou, best)
            labv = jnp.where(upd, lblg, labv)
            mx1 = jnp.where(upd, gx1, mx1)
            my1 = jnp.where(upd, gy1, my1)
            mx2 = jnp.where(upd, gx2, mx2)
            my2 = jnp.where(upd, gy2, my2)

    fgovr = jnp.zeros((ROWS, LANES), jnp.bool_)
    for g in range(G):
        gx1, gy1, gx2, gy2, lblg = gb[g]
        ovr = n == best_inds[g]
        fgovr = fgovr | ovr
        labv = jnp.where(ovr, lblg, labv)
        mx1 = jnp.where(ovr, gx1, mx1)
        my1 = jnp.where(ovr, gy1, my1)
        mx2 = jnp.where(ovr, gx2, mx2)
        my2 = jnp.where(ovr, gy2, my2)

    fg = (best >= FG_T) | fgovr
    ign = (best >= IGN_T) & (~fg)
    wv = jnp.where(ign, 0.0, 1.0)
    labels = jnp.where(fg, labv, 0)

    c0 = cls_ref[0, 0]
    c1 = cls_ref[0, 1]
    c2 = cls_ref[0, 2]
    c3 = cls_ref[0, 3]
    m = jnp.maximum(jnp.maximum(c0, c1), jnp.maximum(c2, c3))
    lse = jnp.log(jnp.exp(c0 - m) + jnp.exp(c1 - m)
                  + jnp.exp(c2 - m) + jnp.exp(c3 - m)) + m
    csel = jnp.where(labels == 0, c0,
                     jnp.where(labels == 1, c1,
                               jnp.where(labels == 2, c2, c3)))
    ce = lse - csel
    cls_loss = jnp.sum(ce * wv) / jnp.maximum(jnp.sum(wv), 1.0)

    gw = mx2 - mx1 + 1.0
    gh = my2 - my1 + 1.0
    gcx = mx1 + 0.5 * gw
    gcy = my1 + 0.5 * gh
    b0 = box_ref[0, 0]
    b1 = box_ref[0, 1]
    b2 = box_ref[0, 2]
    b3 = box_ref[0, 3]
    t0 = ((gcx - rcx) / rw) / STDS[0]
    t1 = ((gcy - rcy) / rh) / STDS[1]
    t2 = jnp.log(gw / rw) / STDS[2]
    t3 = jnp.log(gh / rh) / STDS[3]
    sl1 = jnp.zeros((ROWS, LANES), jnp.float32)
    for bv, tv in ((b0, t0), (b1, t1), (b2, t2), (b3, t3)):
        d = bv - tv
        ad = jnp.abs(d)
        sl1 = sl1 + jnp.where(ad < 1.0, 0.5 * d * d, ad - 0.5)
    fgf = fg.astype(jnp.float32)
    nfg = jnp.maximum(jnp.sum(fgf), 1.0)
    bbox_loss = jnp.sum(sl1 * fgf) / nfg

    d0 = b0 * STDS[0]
    d1 = b1 * STDS[1]
    d2 = b2 * STDS[2]
    d3 = b3 * STDS[3]
    pcx = d0 * rw + rcx
    pcy = d1 * rh + rcy
    pw = jnp.exp(jnp.clip(d2, -4.0, 4.0)) * rw
    ph = jnp.exp(jnp.clip(d3, -4.0, 4.0)) * rh
    px1 = pcx - 0.5 * pw
    py1 = pcy - 0.5 * ph
    px2 = pcx + 0.5 * pw
    py2 = pcy + 0.5 * ph
    iw = jnp.maximum(jnp.minimum(px2, mx2) - jnp.maximum(px1, mx1), 0.0)
    ih = jnp.maximum(jnp.minimum(py2, my2) - jnp.maximum(py1, my1), 0.0)
    inter = iw * ih
    pa = (px2 - px1) * (py2 - py1)
    ga = (mx2 - mx1) * (my2 - my1)
    ious = inter / jnp.maximum(pa + ga - inter, 1e-8)
    iou_loss = jnp.sum((1.0 - ious) * fgf) / nfg

    out_ref[0, i] = cls_loss + bbox_loss + iou_loss


def _run(cls_t, box_t, gts, gt_labels, anchors):
    return pl.pallas_call(
        _body,
        grid=(B,),
        in_specs=[
            pl.BlockSpec((1, 4, ROWS, LANES), lambda i: (i, 0, 0, 0)),
            pl.BlockSpec((1, 4, ROWS, LANES), lambda i: (i, 0, 0, 0)),
            pl.BlockSpec(memory_space=pltpu.SMEM),
            pl.BlockSpec(memory_space=pltpu.SMEM),
            pl.BlockSpec(memory_space=pltpu.SMEM),
        ],
        out_specs=pl.BlockSpec((1, B), lambda i: (0, 0),
                               memory_space=pltpu.SMEM),
        out_shape=jax.ShapeDtypeStruct((1, B), jnp.float32),
    )(cls_t, box_t, gts, gt_labels, anchors)


@jax.jit
def kernel(cls, bbox_2d, gts, anchors, gt_labels):
    cls_t = cls.transpose(0, 2, 1).reshape(B, 4, ROWS, LANES)
    box_t = bbox_2d.transpose(0, 2, 1).reshape(B, 4, ROWS, LANES)
    per_img = _run(cls_t, box_t, gts, gt_labels.astype(jnp.int32), anchors)
    return jnp.mean(per_img)
